# Initial kernel scaffold; baseline (speedup 1.0000x reference)
import jax
import jax.numpy as jnp
from jax.experimental import pallas as pl


def kernel(x, indices, W):
    raise NotImplementedError("write your pallas kernel here")


# speedup vs baseline: 1.9770x; 1.9770x over previous
import functools

import jax
import jax.numpy as jnp
from jax.experimental import pallas as pl
from jax.experimental.pallas import tpu as pltpu


def _fused_kernel(K, idx_ref, x_ref, w_ref, out_ref, xg_ref):
    b = pl.program_id(0)
    e = pl.program_id(1)
    jb = pl.program_id(2)

    @pl.when(jb == 0)
    def _gather():
        def body(k, carry):
            t = idx_ref[b, e, k]
            xg_ref[pl.ds(k, 1), :] = x_ref[0, pl.ds(t, 1), :]
            return carry

        jax.lax.fori_loop(0, K, body, 0, unroll=8)

    out_ref[0, 0] = jnp.dot(
        xg_ref[...], w_ref[0], preferred_element_type=jnp.float32
    )


@functools.partial(jax.jit, static_argnames=("jblk", "interpret"))
def _run(x, indices, W, jblk=1024, interpret=False):
    B, T, I = x.shape
    _, E, K = indices.shape
    J = W.shape[2]
    grid = (B, E, J // jblk)
    grid_spec = pltpu.PrefetchScalarGridSpec(
        num_scalar_prefetch=1,
        grid=grid,
        in_specs=[
            pl.BlockSpec((1, T, I), lambda b, e, jb, idx: (b, 0, 0)),
            pl.BlockSpec((1, I, jblk), lambda b, e, jb, idx: (e, 0, jb)),
        ],
        out_specs=pl.BlockSpec((1, 1, K, jblk), lambda b, e, jb, idx: (b, e, 0, jb)),
        scratch_shapes=[pltpu.VMEM((K, I), jnp.float32)],
    )
    fn = pl.pallas_call(
        functools.partial(_fused_kernel, K),
        grid_spec=grid_spec,
        out_shape=jax.ShapeDtypeStruct((B, E, K, J), jnp.float32),
        compiler_params=pltpu.CompilerParams(
            dimension_semantics=("arbitrary", "arbitrary", "arbitrary"),
        ),
        interpret=interpret,
    )
    return fn(indices, x, W)


def kernel(x, indices, W):
    return _run(x, indices, W)
